# Initial kernel scaffold; baseline (speedup 1.0000x reference)
import jax
import jax.numpy as jnp
from jax.experimental import pallas as pl


def kernel(conf, loc, target):
    raise NotImplementedError("write your pallas kernel here")


# speedup vs baseline: 4.5245x; 4.5245x over previous
import jax
import jax.numpy as jnp
from jax import lax
from jax.experimental import pallas as pl
from jax.experimental.pallas import tpu as pltpu

_B, _C, _A = 32, 81, 8732
_NEG_RATIO = 3
_WEIGHT = 1.0
_INF_BITS = 0x7F800000


def _stage1(conf_ref, loc_ref, tgt_ref, closs_ref, scal_ref):
    b = pl.program_id(0)
    x = conf_ref[0]
    lab = tgt_ref[0, 0:1, :]
    tb = tgt_ref[0, 1:5, :]
    lc = loc_ref[0]

    m = jnp.max(x, axis=0, keepdims=True)
    s = jnp.sum(jnp.exp(x - m), axis=0, keepdims=True)
    lse = jnp.log(s) + m
    cls_i = (lab + 1.0).astype(jnp.int32)
    iota = lax.broadcasted_iota(jnp.int32, (_C, _A), 0)
    picked = jnp.sum(jnp.where(iota == cls_i, x, 0.0), axis=0, keepdims=True)
    closs = lse - picked

    pos = lab > -1.0
    neg = lab == -1.0
    d = jnp.abs(lc - tb)
    sl1 = jnp.where(d < 1.0, 0.5 * d * d, d - 0.5)

    pce_p = jnp.sum(jnp.where(pos, closs, 0.0))
    ll_p = jnp.sum(jnp.where(pos, sl1, 0.0))
    pn_p = jnp.sum(pos.astype(jnp.float32))
    nn_p = jnp.sum(neg.astype(jnp.float32))

    closs_ref[...] = jnp.where(neg, closs, -1.0)[None]

    @pl.when(b == 0)
    def _():
        scal_ref[0] = pce_p
        scal_ref[1] = ll_p
        scal_ref[2] = pn_p
        scal_ref[3] = nn_p

    @pl.when(b != 0)
    def _():
        scal_ref[0] += pce_p
        scal_ref[1] += ll_p
        scal_ref[2] += pn_p
        scal_ref[3] += nn_p


def _stage2(closs_ref, scal_ref, out_ref):
    v = closs_ref[...]
    vb = lax.bitcast_convert_type(v, jnp.int32)
    pos_n = scal_ref[2]
    neg_n = scal_ref[3]
    k = jnp.minimum(neg_n.astype(jnp.int32),
                    _NEG_RATIO * pos_n.astype(jnp.int32))

    def body(_, carry):
        lo, hi = carry
        mid = lo + ((hi - lo) >> 1)
        cnt = jnp.sum((vb >= mid).astype(jnp.int32))
        good = cnt >= k
        return jnp.where(good, mid, lo), jnp.where(good, hi, mid)

    lo, _ = lax.fori_loop(0, 31, body,
                          (jnp.int32(0), jnp.int32(_INF_BITS)))
    gt = vb > lo
    cnt_gt = jnp.sum(gt.astype(jnp.int32))
    sum_gt = jnp.sum(jnp.where(gt, v, 0.0))
    t = lax.bitcast_convert_type(lo, jnp.float32)
    s_top = sum_gt + (k - cnt_gt).astype(jnp.float32) * t
    s_top = jnp.where(k > 0, s_top, 0.0)

    pce = scal_ref[0]
    ll = scal_ref[1]
    denom = pos_n + k.astype(jnp.float32)
    out_ref[0] = (pce + s_top) / denom + _WEIGHT * ll / pos_n


@jax.jit
def kernel(conf, loc, target):
    tgt_t = jnp.transpose(target, (0, 2, 1))
    closs_neg, scal = pl.pallas_call(
        _stage1,
        grid=(_B,),
        in_specs=[
            pl.BlockSpec((1, _C, _A), lambda b: (b, 0, 0)),
            pl.BlockSpec((1, 4, _A), lambda b: (b, 0, 0)),
            pl.BlockSpec((1, 5, _A), lambda b: (b, 0, 0)),
        ],
        out_specs=[
            pl.BlockSpec((1, 1, _A), lambda b: (b, 0, 0)),
            pl.BlockSpec(memory_space=pltpu.SMEM),
        ],
        out_shape=[
            jax.ShapeDtypeStruct((_B, 1, _A), jnp.float32),
            jax.ShapeDtypeStruct((4,), jnp.float32),
        ],
    )(conf, loc, tgt_t)

    loss = pl.pallas_call(
        _stage2,
        in_specs=[
            pl.BlockSpec((_B, 1, _A), lambda: (0, 0, 0)),
            pl.BlockSpec(memory_space=pltpu.SMEM),
        ],
        out_specs=pl.BlockSpec(memory_space=pltpu.SMEM),
        out_shape=jax.ShapeDtypeStruct((1,), jnp.float32),
    )(closs_neg, scal)
    return loss[0]
